# baseline (speedup 1.0000x reference)
import jax
import jax.numpy as jnp
from jax.experimental import pallas as pl
from jax.experimental.pallas import tpu as pltpu

T = 4
BLOCKS = 8
D = 1024
N_TOK = 4096
TN = 1024


def _gating_kernel(g_ref, x_ref, w_ref, o_ref):
    g = g_ref[:]
    m = jnp.max(g, axis=-1, keepdims=True)
    e = jnp.exp(g - m)
    probs = e / jnp.sum(e, axis=-1, keepdims=True)

    x = x_ref[:].astype(jnp.bfloat16)
    for b in range(BLOCKS):
        h = jnp.maximum(
            jnp.dot(x, w_ref[b], preferred_element_type=jnp.float32),
            0.0,
        )
        for t in range(T):
            p = probs[t : t + 1, b : b + 1]
            if b == 0:
                o_ref[t] = p * h
            else:
                o_ref[t] += p * h


def kernel(inputs, W, g_logits):
    w_bf = W.astype(jnp.bfloat16)
    grid = (N_TOK // TN,)
    out = pl.pallas_call(
        _gating_kernel,
        grid=grid,
        in_specs=[
            pl.BlockSpec((T, BLOCKS), lambda n: (0, 0)),
            pl.BlockSpec((TN, D), lambda n: (n, 0)),
            pl.BlockSpec((BLOCKS, D, D), lambda n: (0, 0, 0)),
        ],
        out_specs=pl.BlockSpec((T, TN, D), lambda n: (0, n, 0)),
        out_shape=jax.ShapeDtypeStruct((T, N_TOK, D), jnp.float32),
    )(g_logits, inputs, w_bf)
    return out


# speedup vs baseline: 1.1508x; 1.1508x over previous
import jax
import jax.numpy as jnp
from jax.experimental import pallas as pl
from jax.experimental.pallas import tpu as pltpu

T = 4
BLOCKS = 8
D = 1024
N_TOK = 4096
TN = 1024
DH = D // 2


def _gating_kernel(g_ref, x_ref, w_ref, o_ref):
    g = g_ref[:]
    m = jnp.max(g, axis=-1, keepdims=True)
    e = jnp.exp(g - m)
    probs = e / jnp.sum(e, axis=-1, keepdims=True)

    x = x_ref[:].astype(jnp.bfloat16)
    for b in range(BLOCKS):
        h = jnp.maximum(
            jnp.dot(
                x,
                w_ref[b].astype(jnp.bfloat16),
                preferred_element_type=jnp.float32,
            ),
            0.0,
        )
        for t in range(T):
            p = probs[t : t + 1, b : b + 1]
            if b == 0:
                o_ref[t] = p * h
            else:
                o_ref[t] += p * h


def kernel(inputs, W, g_logits):
    grid = (2, N_TOK // TN)
    out = pl.pallas_call(
        _gating_kernel,
        grid=grid,
        in_specs=[
            pl.BlockSpec((T, BLOCKS), lambda j, n: (0, 0)),
            pl.BlockSpec((TN, D), lambda j, n: (n, 0)),
            pl.BlockSpec((BLOCKS, D, DH), lambda j, n: (0, 0, j)),
        ],
        out_specs=pl.BlockSpec((T, TN, DH), lambda j, n: (0, n, j)),
        out_shape=jax.ShapeDtypeStruct((T, N_TOK, D), jnp.float32),
    )(g_logits, inputs, W)
    return out
